# baseline (speedup 1.0000x reference)
import functools

import jax
import jax.numpy as jnp
from jax import lax
from jax.experimental import pallas as pl
from jax.experimental.pallas import tpu as pltpu
from jax.experimental.pallas import tpu_sc as plsc

_B, _W, _T, _H = 32, 50, 200, 128
_ROW = _T * _H
_NC, _NS, _L = 2, 16, 16
_NW = _NC * _NS


def _sc_body(
    x_hbm, word_hbm, temp_hbm, out_hbm,
    wordbuf, tempbuf, xb0, xb1, is0, is1, os0, os1,
):
    cid = lax.axis_index("c")
    sid = lax.axis_index("s")
    wid = sid * _NC + cid
    base = wid * _W

    pltpu.sync_copy(word_hbm, wordbuf)
    pltpu.sync_copy(temp_hbm, tempbuf)

    bufs = ((xb0, is0, os0), (xb1, is1, os1))

    def start_in(w, xb, ins):
        pltpu.async_copy(x_hbm.at[base + w], xb, ins)

    def wait_in(w, xb, ins):
        pltpu.make_async_copy(x_hbm.at[base + w], xb, ins).wait()

    def start_out(w, xb, outs):
        pltpu.async_copy(xb, out_hbm.at[base + w], outs)

    def wait_out(w, xb, outs):
        pltpu.make_async_copy(xb, out_hbm.at[base + w], outs).wait()

    def compute_row(w, xb):
        wv = tuple(wordbuf[pl.ds(w * _H + j * _L, _L)] for j in range(_H // _L))

        @plsc.parallel_loop(0, _T, step=1, unroll=4)
        def _(t):
            for j in range(_H // _L):
                off = t * _H + j * _L
                plsc.addupdate(
                    xb.at[pl.ds(off, _L)], tempbuf[pl.ds(off, _L)] + wv[j]
                )

    start_in(0, xb0, is0)

    def pair_body(i, carry):
        for p in range(2):
            xb, ins, outs = bufs[p]
            oxb, oins, oouts = bufs[1 - p]
            w = i * 2 + p
            wait_in(w, xb, ins)

            @pl.when(w + 1 < _W)
            def _():
                @pl.when(w >= 1)
                def _():
                    wait_out(w - 1, oxb, oouts)

                start_in(w + 1, oxb, oins)

            compute_row(w, xb)
            start_out(w, xb, outs)
        return carry

    lax.fori_loop(0, _W // 2, pair_body, 0)
    wait_out(_W - 2, xb0, os0)
    wait_out(_W - 1, xb1, os1)


def kernel(x, word_table, temp_table):
    batch, num_words, time_len, hidden = x.shape
    x2 = x.reshape(batch * num_words, time_len * hidden)
    word_flat = word_table.reshape(-1)
    temp_flat = temp_table[:time_len].reshape(-1)

    mesh = plsc.VectorSubcoreMesh(core_axis_name="c", subcore_axis_name="s")
    run = functools.partial(
        pl.kernel,
        mesh=mesh,
        out_type=jax.ShapeDtypeStruct(x2.shape, x2.dtype),
        scratch_types=[
            pltpu.VMEM((num_words * hidden,), jnp.float32),
            pltpu.VMEM((time_len * hidden,), jnp.float32),
            pltpu.VMEM((time_len * hidden,), jnp.float32),
            pltpu.VMEM((time_len * hidden,), jnp.float32),
            pltpu.SemaphoreType.DMA,
            pltpu.SemaphoreType.DMA,
            pltpu.SemaphoreType.DMA,
            pltpu.SemaphoreType.DMA,
        ],
    )(_sc_body)
    out = run(x2, word_flat, temp_flat)
    return out.reshape(x.shape)


# speedup vs baseline: 1.0656x; 1.0656x over previous
import functools

import jax
import jax.numpy as jnp
from jax import lax
from jax.experimental import pallas as pl
from jax.experimental.pallas import tpu as pltpu
from jax.experimental.pallas import tpu_sc as plsc

_B, _W, _T, _H = 32, 50, 200, 128
_ROW = _T * _H
_NC, _NS, _L = 2, 16, 16
_NW = _NC * _NS


def _sc_body(
    x_hbm, word_hbm, temp_hbm, out_hbm,
    wordbuf, tempbuf, xb0, xb1, xb2, is0, is1, is2, os0, os1, os2,
):
    cid = lax.axis_index("c")
    sid = lax.axis_index("s")
    wid = sid * _NC + cid
    base = wid * _W

    pltpu.sync_copy(word_hbm, wordbuf)
    pltpu.sync_copy(temp_hbm, tempbuf)

    bufs = ((xb0, is0, os0), (xb1, is1, os1), (xb2, is2, os2))

    def start_in(w, xb, ins):
        pltpu.async_copy(x_hbm.at[base + w], xb, ins)

    def wait_in(w, xb, ins):
        pltpu.make_async_copy(x_hbm.at[base + w], xb, ins).wait()

    def start_out(w, xb, outs):
        pltpu.async_copy(xb, out_hbm.at[base + w], outs)

    def wait_out(w, xb, outs):
        pltpu.make_async_copy(xb, out_hbm.at[base + w], outs).wait()

    def compute_row(w, xb):
        wv = tuple(wordbuf[pl.ds(w * _H + j * _L, _L)] for j in range(_H // _L))

        @plsc.parallel_loop(0, _T, step=1, unroll=4)
        def _(t):
            for j in range(_H // _L):
                off = t * _H + j * _L
                plsc.addupdate(
                    xb.at[pl.ds(off, _L)], tempbuf[pl.ds(off, _L)] + wv[j]
                )

    start_in(0, xb0, is0)
    start_in(1, xb1, is1)

    def tri_body(i, carry):
        for p in range(3):
            xb, ins, outs = bufs[p]
            nb = bufs[(p + 2) % 3]
            w = i * 3 + p
            wait_in(w, xb, ins)
            compute_row(w, xb)
            start_out(w, xb, outs)

            @pl.when(w >= 1)
            def _():
                wait_out(w - 1, nb[0], nb[2])

            start_in(w + 2, nb[0], nb[1])
        return carry

    lax.fori_loop(0, 16, tri_body, 0)
    for w in (48, 49):
        xb, ins, outs = bufs[w % 3]
        wait_in(w, xb, ins)
        compute_row(w, xb)
        start_out(w, xb, outs)
    for w in (47, 48, 49):
        xb, _, outs = bufs[w % 3]
        wait_out(w, xb, outs)


def kernel(x, word_table, temp_table):
    batch, num_words, time_len, hidden = x.shape
    x2 = x.reshape(batch * num_words, time_len * hidden)
    word_flat = word_table.reshape(-1)
    temp_flat = temp_table[:time_len].reshape(-1)

    mesh = plsc.VectorSubcoreMesh(core_axis_name="c", subcore_axis_name="s")
    run = functools.partial(
        pl.kernel,
        mesh=mesh,
        out_type=jax.ShapeDtypeStruct(x2.shape, x2.dtype),
        scratch_types=[
            pltpu.VMEM((num_words * hidden,), jnp.float32),
            pltpu.VMEM((time_len * hidden,), jnp.float32),
            pltpu.VMEM((time_len * hidden,), jnp.float32),
            pltpu.VMEM((time_len * hidden,), jnp.float32),
            pltpu.VMEM((time_len * hidden,), jnp.float32),
            pltpu.SemaphoreType.DMA,
            pltpu.SemaphoreType.DMA,
            pltpu.SemaphoreType.DMA,
            pltpu.SemaphoreType.DMA,
            pltpu.SemaphoreType.DMA,
            pltpu.SemaphoreType.DMA,
        ],
    )(_sc_body)
    out = run(x2, word_flat, temp_flat)
    return out.reshape(x.shape)


# speedup vs baseline: 3.0969x; 2.9062x over previous
import functools

import jax
import jax.numpy as jnp
from jax import lax
from jax.experimental import pallas as pl
from jax.experimental.pallas import tpu as pltpu
from jax.experimental.pallas import tpu_sc as plsc

_B, _W, _T, _H = 32, 50, 200, 128
_NC, _NS, _L = 2, 16, 16
_NW = _NC * _NS


def _sc_body(
    x_hbm, word_hbm, temp_hbm, out_hbm,
    wordbuf, tempbuf, xb0, xb1, xb2, is0, is1, is2, os0, os1, os2,
):
    cid = lax.axis_index("c")
    sid = lax.axis_index("s")
    wid = sid * _NC + cid
    base = wid * _W

    pltpu.sync_copy(word_hbm, wordbuf)
    pltpu.sync_copy(temp_hbm, tempbuf)

    bufs = ((xb0, is0, os0), (xb1, is1, os1), (xb2, is2, os2))

    def start_in(w, xb, ins):
        pltpu.async_copy(x_hbm.at[pl.ds((base + w) * _T, _T)], xb, ins)

    def wait_in(w, xb, ins):
        pltpu.make_async_copy(x_hbm.at[pl.ds((base + w) * _T, _T)], xb, ins).wait()

    def start_out(w, xb, outs):
        pltpu.async_copy(xb, out_hbm.at[pl.ds((base + w) * _T, _T)], outs)

    def wait_out(w, xb, outs):
        pltpu.make_async_copy(xb, out_hbm.at[pl.ds((base + w) * _T, _T)], outs).wait()

    def compute_row(w, xb):
        wv = tuple(wordbuf[w, pl.ds(j * _L, _L)] for j in range(_H // _L))

        @plsc.parallel_loop(0, _T, step=1, unroll=4)
        def _(t):
            for j in range(_H // _L):
                sl = pl.ds(j * _L, _L)
                plsc.addupdate(xb.at[t, sl], tempbuf[t, sl] + wv[j])

    start_in(0, xb0, is0)
    start_in(1, xb1, is1)

    def tri_body(i, carry):
        for p in range(3):
            xb, ins, outs = bufs[p]
            nb = bufs[(p + 2) % 3]
            w = i * 3 + p
            wait_in(w, xb, ins)
            compute_row(w, xb)
            start_out(w, xb, outs)

            @pl.when(w >= 1)
            def _():
                wait_out(w - 1, nb[0], nb[2])

            start_in(w + 2, nb[0], nb[1])
        return carry

    lax.fori_loop(0, 16, tri_body, 0)
    for w in (48, 49):
        xb, ins, outs = bufs[w % 3]
        wait_in(w, xb, ins)
        compute_row(w, xb)
        start_out(w, xb, outs)
    for w in (47, 48, 49):
        xb, _, outs = bufs[w % 3]
        wait_out(w, xb, outs)


def kernel(x, word_table, temp_table):
    batch, num_words, time_len, hidden = x.shape
    x2 = x.reshape(batch * num_words * time_len, hidden)
    temp_slice = temp_table[:time_len]

    mesh = plsc.VectorSubcoreMesh(core_axis_name="c", subcore_axis_name="s")
    run = functools.partial(
        pl.kernel,
        mesh=mesh,
        out_type=jax.ShapeDtypeStruct(x2.shape, x2.dtype),
        scratch_types=[
            pltpu.VMEM((num_words, hidden), jnp.float32),
            pltpu.VMEM((time_len, hidden), jnp.float32),
            pltpu.VMEM((time_len, hidden), jnp.float32),
            pltpu.VMEM((time_len, hidden), jnp.float32),
            pltpu.VMEM((time_len, hidden), jnp.float32),
            pltpu.SemaphoreType.DMA,
            pltpu.SemaphoreType.DMA,
            pltpu.SemaphoreType.DMA,
            pltpu.SemaphoreType.DMA,
            pltpu.SemaphoreType.DMA,
            pltpu.SemaphoreType.DMA,
        ],
    )(_sc_body)
    out = run(x2, word_table, temp_slice)
    return out.reshape(x.shape)
